# Initial kernel scaffold; baseline (speedup 1.0000x reference)
import jax
import jax.numpy as jnp
from jax.experimental import pallas as pl


def kernel(x, knn_matrix, w1, w2, conv_w, conv_b):
    raise NotImplementedError("write your pallas kernel here")


# speedup vs baseline: 10.4679x; 10.4679x over previous
import functools

import jax
import jax.numpy as jnp
from jax import lax
from jax.experimental import pallas as pl
from jax.experimental.pallas import tpu as pltpu
from jax.experimental.pallas import tpu_sc as plsc

B, N, D, C, K, OUT = 16, 512, 3, 16, 64, 16
BN = B * N
DC = D * C
NC, NS = 2, 16
NW = NC * NS
PW = BN // NW
CP = 8
CPK = CP * K
G = 128
NG = CPK // G
NCH = PW // CP


def _sc_weighted(ptcld, idx, w1t):
    mesh = plsc.VectorSubcoreMesh(core_axis_name="c", subcore_axis_name="s")

    @functools.partial(
        pl.kernel,
        out_type=jax.ShapeDtypeStruct((BN, DC), jnp.float32),
        mesh=mesh,
        scratch_types=[
            pltpu.VMEM((K, C), jnp.float32),
            pltpu.VMEM((CPK,), jnp.int32),
            pltpu.VMEM((CPK, DC), jnp.float32),
            pltpu.VMEM((CP, DC), jnp.float32),
            pltpu.SemaphoreType.DMA,
        ],
        compiler_params=pltpu.CompilerParams(use_tc_tiling_on_sc=False),
    )
    def run(ptcld_hbm, idx_hbm, w1t_hbm, out_hbm, w1_v, idx_v, rows_v, wout_v, sem):
        wid = lax.axis_index("s") * NC + lax.axis_index("c")
        pltpu.sync_copy(w1t_hbm, w1_v)
        point0 = wid * PW

        @pl.loop(0, NCH)
        def _chunk(ch):
            p0 = point0 + ch * CP
            pltpu.sync_copy(idx_hbm.at[pl.ds(p0 * K, CPK)], idx_v)
            copies = [
                pltpu.async_copy(
                    ptcld_hbm.at[idx_v.at[pl.ds(g * G, G)]],
                    rows_v.at[pl.ds(g * G, G)],
                    sem,
                )
                for g in range(NG)
            ]
            for h in copies:
                h.wait()
            for p in range(CP):
                def kstep(k, accs, p=p):
                    a0, a1, a2 = accs
                    wk = w1_v[k, :]
                    r = p * K + k
                    a0 = a0 + rows_v[r, pl.ds(0, 16)] * wk
                    a1 = a1 + rows_v[r, pl.ds(16, 16)] * wk
                    a2 = a2 + rows_v[r, pl.ds(32, 16)] * wk
                    return (a0, a1, a2)

                z = jnp.zeros((16,), jnp.float32)
                a0, a1, a2 = lax.fori_loop(0, K, kstep, (z, z, z))
                wout_v[p, pl.ds(0, 16)] = a0
                wout_v[p, pl.ds(16, 16)] = a1
                wout_v[p, pl.ds(32, 16)] = a2
            pltpu.sync_copy(wout_v, out_hbm.at[pl.ds(p0, CP)])

    return run(ptcld, idx, w1t)


def _tc_finalize(wu, w1, w2):
    def body(wu_ref, w1_ref, w2_ref, o_ref):
        w1m = w1_ref[...]
        s1 = 1.0 / jnp.maximum(
            jnp.sqrt(jnp.sum(w1m * w1m, axis=1, keepdims=True)), 1e-12)
        w2m = w2_ref[...]
        s2 = 1.0 / jnp.maximum(
            jnp.sqrt(jnp.sum(w2m * w2m, axis=0, keepdims=True)), 1e-12)
        wmat = w2m * s2 * s1
        o_ref[...] = jnp.dot(wu_ref[...], wmat,
                             preferred_element_type=jnp.float32)

    return pl.pallas_call(
        body,
        out_shape=jax.ShapeDtypeStruct((BN * D, OUT), jnp.float32),
    )(wu, w1, w2)


def kernel(x, knn_matrix, w1, w2, conv_w, conv_b):
    del conv_w, conv_b
    ptcld = x.reshape(BN, DC)
    offs = (jnp.arange(B, dtype=jnp.int32) * N).reshape(B, 1, 1)
    idx = (knn_matrix.astype(jnp.int32) + offs).reshape(BN * K)
    w1t = jnp.transpose(w1)
    wu = _sc_weighted(ptcld, idx, w1t)
    out = _tc_finalize(wu.reshape(BN * D, C), w1, w2)
    return out.reshape(B, N, D, OUT)


# speedup vs baseline: 11.2890x; 1.0784x over previous
import functools

import jax
import jax.numpy as jnp
from jax import lax
from jax.experimental import pallas as pl
from jax.experimental.pallas import tpu as pltpu
from jax.experimental.pallas import tpu_sc as plsc

B, N, D, C, K, OUT = 16, 512, 3, 16, 64, 16
BN = B * N
DC = D * C
NC, NS = 2, 16
NW = NC * NS
PW = BN // NW
CP = 8
CPK = CP * K
G = 128
NG = CPK // G
NCH = PW // CP


PB = 4


def _sc_weighted(ptcld, knn_local, w1t):
    mesh = plsc.VectorSubcoreMesh(core_axis_name="c", subcore_axis_name="s")

    @functools.partial(
        pl.kernel,
        out_type=jax.ShapeDtypeStruct((BN, DC), jnp.float32),
        mesh=mesh,
        scratch_types=[
            pltpu.VMEM((N, DC), jnp.float32),
            pltpu.VMEM((PW * K,), jnp.int32),
            pltpu.VMEM((K, C), jnp.float32),
            pltpu.VMEM((PW, DC), jnp.float32),
            pltpu.SemaphoreType.DMA,
        ],
        compiler_params=pltpu.CompilerParams(use_tc_tiling_on_sc=False),
    )
    def run(ptcld_hbm, idx_hbm, w1t_hbm, out_hbm, rows_v, idx_v, w1_v, wout_v, sem):
        wid = lax.axis_index("s") * NC + lax.axis_index("c")
        point0 = wid * PW
        batch_row0 = (wid // NC) * N
        copies = [
            pltpu.async_copy(ptcld_hbm.at[pl.ds(batch_row0, N)], rows_v, sem),
            pltpu.async_copy(idx_hbm.at[pl.ds(point0 * K, PW * K)], idx_v, sem),
            pltpu.async_copy(w1t_hbm, w1_v, sem),
        ]
        for h in copies:
            h.wait()

        @pl.loop(0, PW, step=PB)
        def _points(p0):
            ibase = p0 * K
            z = jnp.zeros((16,), jnp.float32)
            acc = [[z, z, z] for _ in range(PB)]
            iv = [None] * PB
            for k in range(K):
                wk = w1_v[k, :]
                if k % 16 == 0:
                    for j in range(PB):
                        iv[j] = idx_v[pl.ds(ibase + j * K + k, 16)]
                for j in range(PB):
                    r = iv[j][k % 16]
                    acc[j][0] = acc[j][0] + rows_v[r, pl.ds(0, 16)] * wk
                    acc[j][1] = acc[j][1] + rows_v[r, pl.ds(16, 16)] * wk
                    acc[j][2] = acc[j][2] + rows_v[r, pl.ds(32, 16)] * wk
            for j in range(PB):
                wout_v[p0 + j, pl.ds(0, 16)] = acc[j][0]
                wout_v[p0 + j, pl.ds(16, 16)] = acc[j][1]
                wout_v[p0 + j, pl.ds(32, 16)] = acc[j][2]

        pltpu.sync_copy(wout_v, out_hbm.at[pl.ds(point0, PW)])

    return run(ptcld, knn_local, w1t)


def _tc_finalize(wu, w1, w2):
    def body(wu_ref, w1_ref, w2_ref, o_ref):
        w1m = w1_ref[...]
        s1 = 1.0 / jnp.maximum(
            jnp.sqrt(jnp.sum(w1m * w1m, axis=1, keepdims=True)), 1e-12)
        w2m = w2_ref[...]
        s2 = 1.0 / jnp.maximum(
            jnp.sqrt(jnp.sum(w2m * w2m, axis=0, keepdims=True)), 1e-12)
        wmat = w2m * s2 * s1
        o_ref[...] = jnp.dot(wu_ref[...], wmat,
                             preferred_element_type=jnp.float32)

    return pl.pallas_call(
        body,
        out_shape=jax.ShapeDtypeStruct((BN * D, OUT), jnp.float32),
    )(wu, w1, w2)


def kernel(x, knn_matrix, w1, w2, conv_w, conv_b):
    del conv_w, conv_b
    ptcld = x.reshape(BN, DC)
    idx = knn_matrix.astype(jnp.int32).reshape(BN * K)
    w1t = jnp.transpose(w1)
    wu = _sc_weighted(ptcld, idx, w1t)
    out = _tc_finalize(wu.reshape(BN * D, C), w1, w2)
    return out.reshape(B, N, D, OUT)


# speedup vs baseline: 15.1287x; 1.3401x over previous
import functools

import jax
import jax.numpy as jnp
from jax import lax
from jax.experimental import pallas as pl
from jax.experimental.pallas import tpu as pltpu
from jax.experimental.pallas import tpu_sc as plsc

B, N, D, C, K, OUT = 16, 512, 3, 16, 64, 16
BN = B * N
DC = D * C
NC, NS = 2, 16
NW = NC * NS
PW = BN // NW
CP = 8
CPK = CP * K
G = 128
NG = CPK // G
NCH = PW // CP


PB = 4


def _sc_weighted(ptcld, knn_local, w1t):
    mesh = plsc.VectorSubcoreMesh(core_axis_name="c", subcore_axis_name="s")

    @functools.partial(
        pl.kernel,
        out_type=jax.ShapeDtypeStruct((BN, DC), jnp.float32),
        mesh=mesh,
        scratch_types=[
            pltpu.VMEM((N * DC,), jnp.float32),
            pltpu.VMEM((PW * K,), jnp.int32),
            pltpu.VMEM((K, C), jnp.float32),
            pltpu.VMEM((PW, DC), jnp.float32),
            pltpu.SemaphoreType.DMA,
        ],
        compiler_params=pltpu.CompilerParams(
            use_tc_tiling_on_sc=False, needs_layout_passes=False),
    )
    def run(ptcld_hbm, idx_hbm, w1t_hbm, out_hbm, rows_v, idx_v, w1_v, wout_v, sem):
        wid = lax.axis_index("s") * NC + lax.axis_index("c")
        point0 = wid * PW
        batch_row0 = (wid // NC) * N
        copies = [
            pltpu.async_copy(
                ptcld_hbm.at[pl.ds(batch_row0 * DC, N * DC)], rows_v, sem),
            pltpu.async_copy(idx_hbm.at[pl.ds(point0 * K, PW * K)], idx_v, sem),
            pltpu.async_copy(w1t_hbm, w1_v, sem),
        ]
        for h in copies:
            h.wait()

        iota = lax.iota(jnp.int32, 16)
        iotas = [iota, iota + 16, iota + 32]

        @pl.loop(0, PW, step=PB)
        def _points(p0):
            ibase = p0 * K
            z = jnp.zeros((16,), jnp.float32)
            acc = [[z, z, z] for _ in range(PB)]
            iv48 = [None] * PB
            for k in range(K):
                wk = w1_v[k, :]
                if k % 16 == 0:
                    for j in range(PB):
                        iv48[j] = idx_v[pl.ds(ibase + j * K + k, 16)] * DC
                for j in range(PB):
                    b48 = jnp.full((16,), iv48[j][k % 16], jnp.int32)
                    for d in range(D):
                        acc[j][d] = acc[j][d] + wk * plsc.load_gather(
                            rows_v, [b48 + iotas[d]])
            for j in range(PB):
                for d in range(D):
                    wout_v[p0 + j, pl.ds(16 * d, 16)] = acc[j][d]

        pltpu.sync_copy(wout_v, out_hbm.at[pl.ds(point0, PW)])

    return run(ptcld, knn_local, w1t)


def _tc_finalize(wu, w1, w2):
    def body(wu_ref, w1_ref, w2_ref, o_ref):
        w1m = w1_ref[...]
        s1 = 1.0 / jnp.maximum(
            jnp.sqrt(jnp.sum(w1m * w1m, axis=1, keepdims=True)), 1e-12)
        w2m = w2_ref[...]
        s2 = 1.0 / jnp.maximum(
            jnp.sqrt(jnp.sum(w2m * w2m, axis=0, keepdims=True)), 1e-12)
        wmat = w2m * s2 * s1
        o_ref[...] = jnp.dot(wu_ref[...], wmat,
                             preferred_element_type=jnp.float32)

    return pl.pallas_call(
        body,
        out_shape=jax.ShapeDtypeStruct((BN * D, OUT), jnp.float32),
    )(wu, w1, w2)


def kernel(x, knn_matrix, w1, w2, conv_w, conv_b):
    del conv_w, conv_b
    ptcld = x.reshape(BN * DC)
    idx = knn_matrix.astype(jnp.int32).reshape(BN * K)
    w1t = jnp.transpose(w1)
    wu = _sc_weighted(ptcld, idx, w1t)
    out = _tc_finalize(wu.reshape(BN * D, C), w1, w2)
    return out.reshape(B, N, D, OUT)


# speedup vs baseline: 18.2459x; 1.2060x over previous
import functools

import jax
import jax.numpy as jnp
from jax import lax
from jax.experimental import pallas as pl
from jax.experimental.pallas import tpu as pltpu
from jax.experimental.pallas import tpu_sc as plsc

B, N, D, C, K, OUT = 16, 512, 3, 16, 64, 16
BN = B * N
DC = D * C
NC, NS = 2, 16
NW = NC * NS
PW = BN // NW
CP = 8
CPK = CP * K
G = 128
NG = CPK // G
NCH = PW // CP


PB = 4


def _sc_weighted(ptcld, knn_local, w1t, wmat):
    mesh = plsc.VectorSubcoreMesh(core_axis_name="c", subcore_axis_name="s")

    @functools.partial(
        pl.kernel,
        out_type=jax.ShapeDtypeStruct((BN, D * OUT), jnp.float32),
        mesh=mesh,
        scratch_types=[
            pltpu.VMEM((N * DC,), jnp.float32),
            pltpu.VMEM((PW * K,), jnp.int32),
            pltpu.VMEM((K, C), jnp.float32),
            pltpu.VMEM((C, OUT), jnp.float32),
            pltpu.VMEM((PW, D * OUT), jnp.float32),
            pltpu.SemaphoreType.DMA,
        ],
        compiler_params=pltpu.CompilerParams(
            use_tc_tiling_on_sc=False, needs_layout_passes=False),
    )
    def run(ptcld_hbm, idx_hbm, w1t_hbm, wmat_hbm, out_hbm,
            rows_v, idx_v, w1_v, wmat_v, wout_v, sem):
        wid = lax.axis_index("s") * NC + lax.axis_index("c")
        point0 = wid * PW
        batch_row0 = (wid // NC) * N
        copies = [
            pltpu.async_copy(
                ptcld_hbm.at[pl.ds(batch_row0 * DC, N * DC)], rows_v, sem),
            pltpu.async_copy(idx_hbm.at[pl.ds(point0 * K, PW * K)], idx_v, sem),
            pltpu.async_copy(w1t_hbm, w1_v, sem),
            pltpu.async_copy(wmat_hbm, wmat_v, sem),
        ]
        for h in copies:
            h.wait()

        iota = lax.iota(jnp.int32, 16)
        iotas = [iota, iota + 16, iota + 32]
        wrow = [wmat_v[c, :] for c in range(C)]

        @pl.loop(0, PW, step=PB)
        def _points(p0):
            ibase = p0 * K
            z = jnp.zeros((16,), jnp.float32)
            acc = [[z, z, z] for _ in range(PB)]
            iv48 = [None] * PB
            for k in range(K):
                wk = w1_v[k, :]
                if k % 16 == 0:
                    for j in range(PB):
                        iv48[j] = idx_v[pl.ds(ibase + j * K + k, 16)] * DC
                for j in range(PB):
                    b48 = jnp.full((16,), iv48[j][k % 16], jnp.int32)
                    for d in range(D):
                        acc[j][d] = acc[j][d] + wk * plsc.load_gather(
                            rows_v, [b48 + iotas[d]])
            for j in range(PB):
                for d in range(D):
                    a = acc[j][d]
                    o = jnp.full((16,), a[0], jnp.float32) * wrow[0]
                    for c in range(1, C):
                        o = o + jnp.full((16,), a[c], jnp.float32) * wrow[c]
                    wout_v[p0 + j, pl.ds(OUT * d, OUT)] = o

        pltpu.sync_copy(wout_v, out_hbm.at[pl.ds(point0, PW)])

    return run(ptcld, knn_local, w1t, wmat)


def _tc_wmat(w1, w2):
    def body(w1_ref, w2_ref, o_ref):
        w1m = w1_ref[...]
        s1 = 1.0 / jnp.maximum(
            jnp.sqrt(jnp.sum(w1m * w1m, axis=1, keepdims=True)), 1e-12)
        w2m = w2_ref[...]
        s2 = 1.0 / jnp.maximum(
            jnp.sqrt(jnp.sum(w2m * w2m, axis=0, keepdims=True)), 1e-12)
        o_ref[...] = w2m * s2 * s1

    return pl.pallas_call(
        body,
        out_shape=jax.ShapeDtypeStruct((C, OUT), jnp.float32),
    )(w1, w2)


def kernel(x, knn_matrix, w1, w2, conv_w, conv_b):
    del conv_w, conv_b
    ptcld = x.reshape(BN * DC)
    idx = knn_matrix.astype(jnp.int32).reshape(BN * K)
    w1t = jnp.transpose(w1)
    wmat = _tc_wmat(w1, w2)
    out = _sc_weighted(ptcld, idx, w1t, wmat)
    return out.reshape(B, N, D, OUT)


# speedup vs baseline: 26.5043x; 1.4526x over previous
import functools

import jax
import jax.numpy as jnp
from jax import lax
from jax.experimental import pallas as pl
from jax.experimental.pallas import tpu as pltpu
from jax.experimental.pallas import tpu_sc as plsc

B, N, D, C, K, OUT = 16, 512, 3, 16, 64, 16
BN = B * N
DC = D * C
NC, NS = 2, 16
NW = NC * NS
PW = BN // NW
CP = 8
CPK = CP * K
G = 128
NG = CPK // G
NCH = PW // CP


PB = 4


def _sc_weighted(ptcld, knn_local, w1t, wmat):
    mesh = plsc.VectorSubcoreMesh(core_axis_name="c", subcore_axis_name="s")

    @functools.partial(
        pl.kernel,
        out_type=jax.ShapeDtypeStruct((BN, D * OUT), jnp.float32),
        mesh=mesh,
        scratch_types=[
            pltpu.VMEM((N, DC), jnp.float32),
            pltpu.VMEM((PW * K,), jnp.int32),
            pltpu.VMEM((K, C), jnp.float32),
            pltpu.VMEM((C, OUT), jnp.float32),
            pltpu.VMEM((PW, D * OUT), jnp.float32),
            pltpu.SemaphoreType.DMA,
        ],
        compiler_params=pltpu.CompilerParams(
            use_tc_tiling_on_sc=False, needs_layout_passes=False),
    )
    def run(ptcld_hbm, idx_hbm, w1t_hbm, wmat_hbm, out_hbm,
            rows_v, idx_v, w1_v, wmat_v, wout_v, sem):
        wid = lax.axis_index("s") * NC + lax.axis_index("c")
        point0 = wid * PW
        batch_row0 = (wid // NC) * N
        copies = [
            pltpu.async_copy(ptcld_hbm.at[pl.ds(batch_row0, N)], rows_v, sem),
            pltpu.async_copy(idx_hbm.at[pl.ds(point0 * K, PW * K)], idx_v, sem),
            pltpu.async_copy(w1t_hbm, w1_v, sem),
            pltpu.async_copy(wmat_hbm, wmat_v, sem),
        ]
        for h in copies:
            h.wait()

        iota = lax.iota(jnp.int32, 16)
        iotas = [iota, iota + 16, iota + 32]
        wrow = [wmat_v[c, :] for c in range(C)]

        @pl.loop(0, PW, step=PB)
        def _points(p0):
            ibase = p0 * K
            z = jnp.zeros((16,), jnp.float32)
            acc = [[z, z, z] for _ in range(PB)]
            iv = [None] * PB
            for k in range(K):
                wk = w1_v[k, :]
                if k % 16 == 0:
                    for j in range(PB):
                        iv[j] = idx_v[pl.ds(ibase + j * K + k, 16)]
                for j in range(PB):
                    b = jnp.full((16,), iv[j][k % 16], jnp.int32)
                    for d in range(D):
                        acc[j][d] = acc[j][d] + wk * plsc.load_gather(
                            rows_v, [b, iotas[d]])
            for j in range(PB):
                for d in range(D):
                    a = acc[j][d]
                    o = jnp.full((16,), a[0], jnp.float32) * wrow[0]
                    for c in range(1, C):
                        o = o + jnp.full((16,), a[c], jnp.float32) * wrow[c]
                    wout_v[p0 + j, pl.ds(OUT * d, OUT)] = o

        pltpu.sync_copy(wout_v, out_hbm.at[pl.ds(point0, PW)])

    return run(ptcld, knn_local, w1t, wmat)


def _tc_wmat(w1, w2):
    def body(w1_ref, w2_ref, o_ref):
        w1m = w1_ref[...]
        s1 = 1.0 / jnp.maximum(
            jnp.sqrt(jnp.sum(w1m * w1m, axis=1, keepdims=True)), 1e-12)
        w2m = w2_ref[...]
        s2 = 1.0 / jnp.maximum(
            jnp.sqrt(jnp.sum(w2m * w2m, axis=0, keepdims=True)), 1e-12)
        o_ref[...] = w2m * s2 * s1

    return pl.pallas_call(
        body,
        out_shape=jax.ShapeDtypeStruct((C, OUT), jnp.float32),
    )(w1, w2)


def kernel(x, knn_matrix, w1, w2, conv_w, conv_b):
    del conv_w, conv_b
    ptcld = x.reshape(BN, DC)
    idx = knn_matrix.astype(jnp.int32).reshape(BN * K)
    w1t = jnp.transpose(w1)
    wmat = _tc_wmat(w1, w2)
    out = _sc_weighted(ptcld, idx, w1t, wmat)
    return out.reshape(B, N, D, OUT)
